# baseline (speedup 1.0000x reference)
import functools

import jax
import jax.numpy as jnp
from jax import lax
from jax.experimental import pallas as pl
from jax.experimental.pallas import tpu as pltpu


def _fused_deform_kernel(xp_ref, wt_ref, b_ref, wf_ref, out_ref,
                         x4_ref, off_ref, idxT_ref, gw_ref, g_ref,
                         idx_smem, dma_sem, *, H, W, C, OUTC, UNROLL):
    Hp, Wp = H + 2, W + 2
    P = Hp * Wp
    HW = H * W
    N = 9
    f32 = jnp.float32

    x = xp_ref[0]
    acc = jnp.zeros((HW, off_ref.shape[1]), f32)
    for kr in range(3):
        for kc in range(3):
            patch = x[kr:kr + H, kc:kc + W, :].reshape(HW, C)
            acc = acc + jnp.dot(patch, wt_ref[kr * 3 + kc],
                                preferred_element_type=f32)
    off_ref[...] = acc + b_ref[...]

    xflat = x.reshape(P, C)
    x4_ref[:, 0:C] = xflat
    x4_ref[0:P - 1, C:2 * C] = xflat[1:P]
    x4_ref[0:P - Wp, 2 * C:3 * C] = xflat[Wp:P]
    x4_ref[0:P - Wp - 1, 3 * C:4 * C] = xflat[Wp + 1:P]

    off_r = off_ref[:, 0:N]
    off_c = off_ref[:, N:2 * N]
    t_idx = lax.broadcasted_iota(jnp.int32, (HW, 1), 0)
    h_i = t_idx // W
    w_i = t_idx - h_i * W
    n_i = lax.broadcasted_iota(jnp.int32, (1, N), 1)
    pn_r = (n_i // 3 - 1).astype(f32)
    pn_c = (n_i % 3 - 1).astype(f32)

    p_r = h_i.astype(f32) + 1.0 + pn_r + off_r
    p_c = w_i.astype(f32) + 1.0 + pn_c + off_c
    fr = jnp.floor(p_r)
    fc = jnp.floor(p_c)
    pr = jnp.clip(p_r, 0.0, Hp - 1.0)
    pc = jnp.clip(p_c, 0.0, Wp - 1.0)
    q0r = jnp.clip(fr, 0.0, Hp - 1.0)
    q1r = jnp.clip(fr + 1.0, 0.0, Hp - 1.0)
    q0c = jnp.clip(fc, 0.0, Wp - 1.0)
    q1c = jnp.clip(fc + 1.0, 0.0, Wp - 1.0)
    r0 = jnp.clip(fr, 0.0, Hp - 2.0)
    c0 = jnp.clip(fc, 0.0, Wp - 2.0)

    a0r = 1.0 + (q0r - pr)
    a1r = 1.0 - (q1r - pr)
    a0c = 1.0 + (q0c - pc)
    a1c = 1.0 - (q1c - pc)
    zero = jnp.zeros_like(a0r)
    w_top = (jnp.where(q0r == r0, a0r, zero)
             + jnp.where(q1r == r0, a1r, zero))
    w_bot = (jnp.where(q0r == r0 + 1.0, a0r, zero)
             + jnp.where(q1r == r0 + 1.0, a1r, zero))
    w_lft = (jnp.where(q0c == c0, a0c, zero)
             + jnp.where(q1c == c0, a1c, zero))
    w_rgt = (jnp.where(q0c == c0 + 1.0, a0c, zero)
             + jnp.where(q1c == c0 + 1.0, a1c, zero))

    gw_ref[:, 0:N] = w_top * w_lft
    gw_ref[:, 16:16 + N] = w_top * w_rgt
    gw_ref[:, 32:32 + N] = w_bot * w_lft
    gw_ref[:, 48:48 + N] = w_bot * w_rgt

    idx = (r0 * Wp + c0).astype(jnp.int32)
    idxT_ref[0:N, :] = jnp.transpose(idx)
    cp = pltpu.make_async_copy(idxT_ref, idx_smem, dma_sem)
    cp.start()
    cp.wait()

    conv = jnp.zeros((HW, OUTC), f32)
    for n in range(N):
        def gbody(j, _, n=n):
            tb = j * UNROLL
            for u in range(UNROLL):
                g_ref[tb + u, :] = x4_ref[idx_smem[n, tb + u], :]
            return 0
        lax.fori_loop(0, HW // UNROLL, gbody, 0)
        g = g_ref[...]
        xq = (gw_ref[:, n:n + 1] * g[:, 0:C]
              + gw_ref[:, 16 + n:17 + n] * g[:, C:2 * C]
              + gw_ref[:, 32 + n:33 + n] * g[:, 2 * C:3 * C]
              + gw_ref[:, 48 + n:49 + n] * g[:, 3 * C:4 * C])
        conv = conv + jnp.dot(xq, wf_ref[n], preferred_element_type=f32)
    out_ref[0] = conv


def _deform_conv2d(x, w_off, b_off, w_conv):
    B, C, H, W = x.shape
    Hp, Wp = H + 2, W + 2
    P = Hp * Wp
    HW = H * W
    N = 9
    outc = w_conv.shape[0]
    OFFPAD = 128
    UNROLL = 8

    x_nhwc = jnp.transpose(x, (0, 2, 3, 1)).astype(jnp.float32)
    xp = jnp.pad(x_nhwc, ((0, 0), (1, 1), (1, 1), (0, 0)))

    w_taps = jnp.transpose(w_off, (2, 3, 1, 0)).reshape(9, C, 2 * N)
    w_taps = jnp.pad(w_taps, ((0, 0), (0, 0), (0, OFFPAD - 2 * N)))
    w_taps = w_taps.astype(jnp.float32)
    b_pad = jnp.pad(b_off.astype(jnp.float32),
                    (0, OFFPAD - 2 * N)).reshape(1, OFFPAD)
    wf = jnp.transpose(w_conv, (2, 3, 1, 0)).reshape(N, C, outc)
    wf = wf.astype(jnp.float32)

    cost = pl.CostEstimate(
        flops=2 * B * HW * C * (9 * OFFPAD + N * outc),
        transcendentals=0,
        bytes_accessed=4 * B * (Hp * Wp * C + HW * outc),
    )

    out = pl.pallas_call(
        functools.partial(_fused_deform_kernel, H=H, W=W, C=C, OUTC=outc,
                          UNROLL=UNROLL),
        out_shape=jax.ShapeDtypeStruct((B, HW, outc), jnp.float32),
        grid=(B,),
        in_specs=[
            pl.BlockSpec((1, Hp, Wp, C), lambda b: (b, 0, 0, 0)),
            pl.BlockSpec((9, C, OFFPAD), lambda b: (0, 0, 0)),
            pl.BlockSpec((1, OFFPAD), lambda b: (0, 0)),
            pl.BlockSpec((N, C, outc), lambda b: (0, 0, 0)),
        ],
        out_specs=pl.BlockSpec((1, HW, outc), lambda b: (b, 0, 0)),
        scratch_shapes=[
            pltpu.VMEM((P, 4 * C), jnp.float32),
            pltpu.VMEM((HW, OFFPAD), jnp.float32),
            pltpu.VMEM((16, HW), jnp.int32),
            pltpu.VMEM((HW, 64), jnp.float32),
            pltpu.VMEM((HW, 4 * C), jnp.float32),
            pltpu.SMEM((16, HW), jnp.int32),
            pltpu.SemaphoreType.DMA,
        ],
        compiler_params=pltpu.CompilerParams(
            dimension_semantics=("parallel",),
            vmem_limit_bytes=100 * 1024 * 1024,
        ),
        cost_estimate=cost,
    )(xp, w_taps, b_pad, wf)

    out = out.reshape(B, H, W, outc)
    return jnp.transpose(out, (0, 3, 1, 2))


def kernel(x, w_off, b_off, w_conv):
    return _deform_conv2d(x, w_off, b_off, w_conv)


# speedup vs baseline: 1.7537x; 1.7537x over previous
import functools

import jax
import jax.numpy as jnp
from jax import lax
from jax.experimental import pallas as pl
from jax.experimental.pallas import tpu as pltpu


def _fused_deform_kernel(xp_ref, wt_ref, b_ref, e_ref, wf_ref, out_ref,
                         x4_ref, off_ref, idxT_ref, gw_ref, g_ref,
                         idx_smem, dma_sem, *, H, W, C, OUTC, UNROLL):
    Hp, Wp = H + 2, W + 2
    P = Hp * Wp
    HW = H * W
    N = 9
    f32 = jnp.float32

    x = xp_ref[0]
    acc = jnp.zeros((HW, off_ref.shape[1]), f32)
    for kr in range(3):
        for kc in range(3):
            patch = x[kr:kr + H, kc:kc + W, :].reshape(HW, C)
            acc = acc + jnp.dot(patch, wt_ref[kr * 3 + kc],
                                preferred_element_type=f32)
    off_ref[...] = acc + b_ref[...]

    xflat = x.reshape(P, C)
    x4_ref[:, 0, 0:C] = xflat
    x4_ref[0:P - 1, 0, C:2 * C] = xflat[1:P]
    x4_ref[0:P - Wp, 0, 2 * C:3 * C] = xflat[Wp:P]
    x4_ref[0:P - Wp - 1, 0, 3 * C:4 * C] = xflat[Wp + 1:P]

    off_r = off_ref[:, 0:N]
    off_c = off_ref[:, N:2 * N]
    t_idx = lax.broadcasted_iota(jnp.int32, (HW, 1), 0)
    h_i = t_idx // W
    w_i = t_idx - h_i * W
    n_i = lax.broadcasted_iota(jnp.int32, (1, N), 1)
    pn_r = (n_i // 3 - 1).astype(f32)
    pn_c = (n_i % 3 - 1).astype(f32)

    p_r = h_i.astype(f32) + 1.0 + pn_r + off_r
    p_c = w_i.astype(f32) + 1.0 + pn_c + off_c
    fr = jnp.floor(p_r)
    fc = jnp.floor(p_c)
    pr = jnp.clip(p_r, 0.0, Hp - 1.0)
    pc = jnp.clip(p_c, 0.0, Wp - 1.0)
    q0r = jnp.clip(fr, 0.0, Hp - 1.0)
    q1r = jnp.clip(fr + 1.0, 0.0, Hp - 1.0)
    q0c = jnp.clip(fc, 0.0, Wp - 1.0)
    q1c = jnp.clip(fc + 1.0, 0.0, Wp - 1.0)
    r0 = jnp.clip(fr, 0.0, Hp - 2.0)
    c0 = jnp.clip(fc, 0.0, Wp - 2.0)

    a0r = 1.0 + (q0r - pr)
    a1r = 1.0 - (q1r - pr)
    a0c = 1.0 + (q0c - pc)
    a1c = 1.0 - (q1c - pc)
    zero = jnp.zeros_like(a0r)
    w_top = (jnp.where(q0r == r0, a0r, zero)
             + jnp.where(q1r == r0, a1r, zero))
    w_bot = (jnp.where(q0r == r0 + 1.0, a0r, zero)
             + jnp.where(q1r == r0 + 1.0, a1r, zero))
    w_lft = (jnp.where(q0c == c0, a0c, zero)
             + jnp.where(q1c == c0, a1c, zero))
    w_rgt = (jnp.where(q0c == c0 + 1.0, a0c, zero)
             + jnp.where(q1c == c0 + 1.0, a1c, zero))

    gw_ref[...] = jnp.zeros(gw_ref.shape, f32)
    gw_ref[:, 0:N] = w_top * w_lft
    gw_ref[:, 16:16 + N] = w_top * w_rgt
    gw_ref[:, 32:32 + N] = w_bot * w_lft
    gw_ref[:, 48:48 + N] = w_bot * w_rgt

    idx = (r0 * Wp + c0).astype(jnp.int32)
    idxT_ref[0:N, :] = jnp.transpose(idx)
    cp = pltpu.make_async_copy(idxT_ref, idx_smem, dma_sem)
    cp.start()
    cp.wait()

    for n in range(N):
        def gbody(j, _, n=n):
            tb = j * UNROLL
            for u in range(UNROLL):
                g_ref[j, u, n * 4 * C:(n + 1) * 4 * C] = (
                    x4_ref[idx_smem[n, tb + u], 0, :])
            return 0
        lax.fori_loop(0, HW // UNROLL, gbody, 0)

    gww = jnp.dot(gw_ref[...], e_ref[...], preferred_element_type=f32)
    gwide = g_ref[...].reshape(HW, N * 4 * C)
    out_ref[0] = jnp.dot(gww * gwide, wf_ref[...],
                         preferred_element_type=f32)


def _deform_conv2d(x, w_off, b_off, w_conv):
    B, C, H, W = x.shape
    Hp, Wp = H + 2, W + 2
    P = Hp * Wp
    HW = H * W
    N = 9
    outc = w_conv.shape[0]
    OFFPAD = 128
    UNROLL = 16

    x_nhwc = jnp.transpose(x, (0, 2, 3, 1)).astype(jnp.float32)
    xp = jnp.pad(x_nhwc, ((0, 0), (1, 1), (1, 1), (0, 0)))

    w_taps = jnp.transpose(w_off, (2, 3, 1, 0)).reshape(9, C, 2 * N)
    w_taps = jnp.pad(w_taps, ((0, 0), (0, 0), (0, OFFPAD - 2 * N)))
    w_taps = w_taps.astype(jnp.float32)
    b_pad = jnp.pad(b_off.astype(jnp.float32),
                    (0, OFFPAD - 2 * N)).reshape(1, OFFPAD)
    wf = jnp.transpose(w_conv, (2, 3, 1, 0)).reshape(N, 1, C, outc)
    wf4 = jnp.broadcast_to(wf, (N, 4, C, outc)).reshape(N * 4 * C, outc)
    wf4 = wf4.astype(jnp.float32)
    j_i = jnp.arange(64)[:, None]
    m_i = jnp.arange(N * 4 * C)[None, :]
    n_of = m_i // (4 * C)
    k_of = (m_i % (4 * C)) // C
    emat = (j_i == (k_of * 16 + n_of)).astype(jnp.float32)

    cost = pl.CostEstimate(
        flops=2 * B * HW * C * (9 * OFFPAD + N * outc),
        transcendentals=0,
        bytes_accessed=4 * B * (Hp * Wp * C + HW * outc),
    )

    out = pl.pallas_call(
        functools.partial(_fused_deform_kernel, H=H, W=W, C=C, OUTC=outc,
                          UNROLL=UNROLL),
        out_shape=jax.ShapeDtypeStruct((B, HW, outc), jnp.float32),
        grid=(B,),
        in_specs=[
            pl.BlockSpec((1, Hp, Wp, C), lambda b: (b, 0, 0, 0)),
            pl.BlockSpec((9, C, OFFPAD), lambda b: (0, 0, 0)),
            pl.BlockSpec((1, OFFPAD), lambda b: (0, 0)),
            pl.BlockSpec((64, N * 4 * C), lambda b: (0, 0)),
            pl.BlockSpec((N * 4 * C, outc), lambda b: (0, 0)),
        ],
        out_specs=pl.BlockSpec((1, HW, outc), lambda b: (b, 0, 0)),
        scratch_shapes=[
            pltpu.VMEM((P, 1, 4 * C), jnp.float32),
            pltpu.VMEM((HW, OFFPAD), jnp.float32),
            pltpu.VMEM((16, HW), jnp.int32),
            pltpu.VMEM((HW, 64), jnp.float32),
            pltpu.VMEM((HW // UNROLL, UNROLL, N * 4 * C), jnp.float32),
            pltpu.SMEM((16, HW), jnp.int32),
            pltpu.SemaphoreType.DMA,
        ],
        compiler_params=pltpu.CompilerParams(
            dimension_semantics=("parallel",),
            vmem_limit_bytes=100 * 1024 * 1024,
        ),
        cost_estimate=cost,
    )(xp, w_taps, b_pad, emat, wf4)

    out = out.reshape(B, H, W, outc)
    return jnp.transpose(out, (0, 3, 1, 2))


def kernel(x, w_off, b_off, w_conv):
    return _deform_conv2d(x, w_off, b_off, w_conv)


# speedup vs baseline: 1.8381x; 1.0482x over previous
import functools

import jax
import jax.numpy as jnp
from jax import lax
from jax.experimental import pallas as pl
from jax.experimental.pallas import tpu as pltpu


def _fused_deform_kernel(xp_ref, wt_ref, b_ref, e_ref, wf_ref, out_ref,
                         x4_ref, off_ref, idxT_ref, gw_ref, g_ref,
                         idx_smem, dma_sem, *, H, W, C, OUTC, UNROLL):
    Hp, Wp = H + 2, W + 2
    P = Hp * Wp
    HW = H * W
    N = 9
    f32 = jnp.float32

    x = xp_ref[0]
    acc = jnp.zeros((HW, off_ref.shape[1]), f32)
    for kr in range(3):
        for kc in range(3):
            patch = x[kr:kr + H, kc:kc + W, :].reshape(HW, C)
            acc = acc + jnp.dot(patch, wt_ref[kr * 3 + kc],
                                preferred_element_type=f32)
    off_ref[...] = acc + b_ref[...]

    xflat = x.reshape(P, C)
    x4_ref[:, 0, 0:C] = xflat
    x4_ref[0:P - 1, 0, C:2 * C] = xflat[1:P]
    x4_ref[0:P - Wp, 0, 2 * C:3 * C] = xflat[Wp:P]
    x4_ref[0:P - Wp - 1, 0, 3 * C:4 * C] = xflat[Wp + 1:P]

    off_r = off_ref[:, 0:N]
    off_c = off_ref[:, N:2 * N]
    t_idx = lax.broadcasted_iota(jnp.int32, (HW, 1), 0)
    h_i = t_idx // W
    w_i = t_idx - h_i * W
    n_i = lax.broadcasted_iota(jnp.int32, (1, N), 1)
    pn_r = (n_i // 3 - 1).astype(f32)
    pn_c = (n_i % 3 - 1).astype(f32)

    p_r = h_i.astype(f32) + 1.0 + pn_r + off_r
    p_c = w_i.astype(f32) + 1.0 + pn_c + off_c
    fr = jnp.floor(p_r)
    fc = jnp.floor(p_c)
    pr = jnp.clip(p_r, 0.0, Hp - 1.0)
    pc = jnp.clip(p_c, 0.0, Wp - 1.0)
    q0r = jnp.clip(fr, 0.0, Hp - 1.0)
    q1r = jnp.clip(fr + 1.0, 0.0, Hp - 1.0)
    q0c = jnp.clip(fc, 0.0, Wp - 1.0)
    q1c = jnp.clip(fc + 1.0, 0.0, Wp - 1.0)
    r0 = jnp.clip(fr, 0.0, Hp - 2.0)
    c0 = jnp.clip(fc, 0.0, Wp - 2.0)

    a0r = 1.0 + (q0r - pr)
    a1r = 1.0 - (q1r - pr)
    a0c = 1.0 + (q0c - pc)
    a1c = 1.0 - (q1c - pc)
    zero = jnp.zeros_like(a0r)
    w_top = (jnp.where(q0r == r0, a0r, zero)
             + jnp.where(q1r == r0, a1r, zero))
    w_bot = (jnp.where(q0r == r0 + 1.0, a0r, zero)
             + jnp.where(q1r == r0 + 1.0, a1r, zero))
    w_lft = (jnp.where(q0c == c0, a0c, zero)
             + jnp.where(q1c == c0, a1c, zero))
    w_rgt = (jnp.where(q0c == c0 + 1.0, a0c, zero)
             + jnp.where(q1c == c0 + 1.0, a1c, zero))

    gw_ref[...] = jnp.zeros(gw_ref.shape, f32)
    gw_ref[:, 0:N] = w_top * w_lft
    gw_ref[:, 16:16 + N] = w_top * w_rgt
    gw_ref[:, 32:32 + N] = w_bot * w_lft
    gw_ref[:, 48:48 + N] = w_bot * w_rgt

    idx = (r0 * Wp + c0).astype(jnp.int32)
    idxT_ref[0:N, :] = jnp.transpose(idx)
    cp = pltpu.make_async_copy(idxT_ref, idx_smem, dma_sem)
    cp.start()
    cp.wait()

    for n in range(N):
        def gbody(j, _, n=n):
            tb = j * UNROLL
            vals = [x4_ref[idx_smem[n, tb + u], 0, :] for u in range(UNROLL)]
            for u in range(UNROLL):
                g_ref[j, u, n * 4 * C:(n + 1) * 4 * C] = vals[u]
            return 0
        lax.fori_loop(0, HW // UNROLL, gbody, 0)

    gww = jnp.dot(gw_ref[...], e_ref[...], preferred_element_type=f32)
    gwide = g_ref[...].reshape(HW, N * 4 * C)
    out_ref[0] = jnp.dot(gww * gwide, wf_ref[...],
                         preferred_element_type=f32)


def _deform_conv2d(x, w_off, b_off, w_conv):
    B, C, H, W = x.shape
    Hp, Wp = H + 2, W + 2
    P = Hp * Wp
    HW = H * W
    N = 9
    outc = w_conv.shape[0]
    OFFPAD = 128
    UNROLL = 32

    x_nhwc = jnp.transpose(x, (0, 2, 3, 1)).astype(jnp.float32)
    xp = jnp.pad(x_nhwc, ((0, 0), (1, 1), (1, 1), (0, 0)))

    w_taps = jnp.transpose(w_off, (2, 3, 1, 0)).reshape(9, C, 2 * N)
    w_taps = jnp.pad(w_taps, ((0, 0), (0, 0), (0, OFFPAD - 2 * N)))
    w_taps = w_taps.astype(jnp.float32)
    b_pad = jnp.pad(b_off.astype(jnp.float32),
                    (0, OFFPAD - 2 * N)).reshape(1, OFFPAD)
    wf = jnp.transpose(w_conv, (2, 3, 1, 0)).reshape(N, 1, C, outc)
    wf4 = jnp.broadcast_to(wf, (N, 4, C, outc)).reshape(N * 4 * C, outc)
    wf4 = wf4.astype(jnp.float32)
    j_i = jnp.arange(64)[:, None]
    m_i = jnp.arange(N * 4 * C)[None, :]
    n_of = m_i // (4 * C)
    k_of = (m_i % (4 * C)) // C
    emat = (j_i == (k_of * 16 + n_of)).astype(jnp.float32)

    cost = pl.CostEstimate(
        flops=2 * B * HW * C * (9 * OFFPAD + N * outc),
        transcendentals=0,
        bytes_accessed=4 * B * (Hp * Wp * C + HW * outc),
    )

    out = pl.pallas_call(
        functools.partial(_fused_deform_kernel, H=H, W=W, C=C, OUTC=outc,
                          UNROLL=UNROLL),
        out_shape=jax.ShapeDtypeStruct((B, HW, outc), jnp.float32),
        grid=(B,),
        in_specs=[
            pl.BlockSpec((1, Hp, Wp, C), lambda b: (b, 0, 0, 0)),
            pl.BlockSpec((9, C, OFFPAD), lambda b: (0, 0, 0)),
            pl.BlockSpec((1, OFFPAD), lambda b: (0, 0)),
            pl.BlockSpec((64, N * 4 * C), lambda b: (0, 0)),
            pl.BlockSpec((N * 4 * C, outc), lambda b: (0, 0)),
        ],
        out_specs=pl.BlockSpec((1, HW, outc), lambda b: (b, 0, 0)),
        scratch_shapes=[
            pltpu.VMEM((P, 1, 4 * C), jnp.float32),
            pltpu.VMEM((HW, OFFPAD), jnp.float32),
            pltpu.VMEM((16, HW), jnp.int32),
            pltpu.VMEM((HW, 64), jnp.float32),
            pltpu.VMEM((HW // UNROLL, UNROLL, N * 4 * C), jnp.float32),
            pltpu.SMEM((16, HW), jnp.int32),
            pltpu.SemaphoreType.DMA,
        ],
        compiler_params=pltpu.CompilerParams(
            dimension_semantics=("parallel",),
            vmem_limit_bytes=100 * 1024 * 1024,
        ),
        cost_estimate=cost,
    )(xp, w_taps, b_pad, emat, wf4)

    out = out.reshape(B, H, W, outc)
    return jnp.transpose(out, (0, 3, 1, 2))


def kernel(x, w_off, b_off, w_conv):
    return _deform_conv2d(x, w_off, b_off, w_conv)


# speedup vs baseline: 2.8657x; 1.5591x over previous
import functools

import jax
import jax.numpy as jnp
from jax import lax
from jax.experimental import pallas as pl
from jax.experimental.pallas import tpu as pltpu


def _fused_deform_kernel(xp_ref, wt_ref, b_ref, e_ref, wf_ref, out_ref,
                         x4_ref, off_ref, idxT_ref, gw_ref, g_ref,
                         idx_smem, dma_sem, *, H, W, C, OUTC, UNROLL):
    Hp, Wp = H + 2, W + 2
    P = Hp * Wp
    HW = H * W
    N = 9
    f32 = jnp.float32

    x = xp_ref[0]
    acc = jnp.zeros((HW, off_ref.shape[1]), f32)
    for kr in range(3):
        for kc in range(3):
            patch = x[kr:kr + H, kc:kc + W, :].reshape(HW, C)
            acc = acc + jnp.dot(patch, wt_ref[kr * 3 + kc],
                                preferred_element_type=f32)
    off_ref[...] = acc + b_ref[...]

    xflat = x.reshape(P, C)
    x4_ref[:, 0, 0:C] = xflat
    x4_ref[0:P - 1, 0, C:2 * C] = xflat[1:P]
    x4_ref[0:P - Wp, 0, 2 * C:3 * C] = xflat[Wp:P]
    x4_ref[0:P - Wp - 1, 0, 3 * C:4 * C] = xflat[Wp + 1:P]

    off_r = off_ref[:, 0:N]
    off_c = off_ref[:, N:2 * N]
    t_idx = lax.broadcasted_iota(jnp.int32, (HW, 1), 0)
    h_i = t_idx // W
    w_i = t_idx - h_i * W
    n_i = lax.broadcasted_iota(jnp.int32, (1, N), 1)
    pn_r = (n_i // 3 - 1).astype(f32)
    pn_c = (n_i % 3 - 1).astype(f32)

    p_r = h_i.astype(f32) + 1.0 + pn_r + off_r
    p_c = w_i.astype(f32) + 1.0 + pn_c + off_c
    fr = jnp.floor(p_r)
    fc = jnp.floor(p_c)
    pr = jnp.clip(p_r, 0.0, Hp - 1.0)
    pc = jnp.clip(p_c, 0.0, Wp - 1.0)
    q0r = jnp.clip(fr, 0.0, Hp - 1.0)
    q1r = jnp.clip(fr + 1.0, 0.0, Hp - 1.0)
    q0c = jnp.clip(fc, 0.0, Wp - 1.0)
    q1c = jnp.clip(fc + 1.0, 0.0, Wp - 1.0)
    r0 = jnp.clip(fr, 0.0, Hp - 2.0)
    c0 = jnp.clip(fc, 0.0, Wp - 2.0)

    a0r = 1.0 + (q0r - pr)
    a1r = 1.0 - (q1r - pr)
    a0c = 1.0 + (q0c - pc)
    a1c = 1.0 - (q1c - pc)
    zero = jnp.zeros_like(a0r)
    w_top = (jnp.where(q0r == r0, a0r, zero)
             + jnp.where(q1r == r0, a1r, zero))
    w_bot = (jnp.where(q0r == r0 + 1.0, a0r, zero)
             + jnp.where(q1r == r0 + 1.0, a1r, zero))
    w_lft = (jnp.where(q0c == c0, a0c, zero)
             + jnp.where(q1c == c0, a1c, zero))
    w_rgt = (jnp.where(q0c == c0 + 1.0, a0c, zero)
             + jnp.where(q1c == c0 + 1.0, a1c, zero))

    gw_ref[...] = jnp.zeros(gw_ref.shape, f32)
    gw_ref[:, 0:N] = w_top * w_lft
    gw_ref[:, 16:16 + N] = w_top * w_rgt
    gw_ref[:, 32:32 + N] = w_bot * w_lft
    gw_ref[:, 48:48 + N] = w_bot * w_rgt

    idx = (r0 * Wp + c0).astype(jnp.int32)
    idxT_ref[0:N, :] = jnp.transpose(idx)
    for nn in range(N):
        pltpu.make_async_copy(idxT_ref.at[nn],
                              idx_smem.at[pl.ds(nn * HW, HW)], dma_sem).start()
    for nn in range(N):
        pltpu.make_async_copy(idxT_ref.at[nn],
                              idx_smem.at[pl.ds(nn * HW, HW)], dma_sem).wait()

    for n in range(N):
        def gbody(j, _, n=n):
            tb = j * UNROLL
            vals = [x4_ref[idx_smem[n * HW + tb + u], 0, :]
                    for u in range(UNROLL)]
            for u in range(UNROLL):
                g_ref[j, u, n * 4 * C:(n + 1) * 4 * C] = vals[u]
            return 0
        lax.fori_loop(0, HW // UNROLL, gbody, 0)

    gww = jnp.dot(gw_ref[...], e_ref[...], preferred_element_type=f32)
    gwide = g_ref[...].reshape(HW, N * 4 * C)
    out_ref[0] = jnp.dot(gww * gwide, wf_ref[...],
                         preferred_element_type=f32)


def _deform_conv2d(x, w_off, b_off, w_conv):
    B, C, H, W = x.shape
    Hp, Wp = H + 2, W + 2
    P = Hp * Wp
    HW = H * W
    N = 9
    outc = w_conv.shape[0]
    OFFPAD = 128
    UNROLL = 32

    x_nhwc = jnp.transpose(x, (0, 2, 3, 1)).astype(jnp.float32)
    xp = jnp.pad(x_nhwc, ((0, 0), (1, 1), (1, 1), (0, 0)))

    w_taps = jnp.transpose(w_off, (2, 3, 1, 0)).reshape(9, C, 2 * N)
    w_taps = jnp.pad(w_taps, ((0, 0), (0, 0), (0, OFFPAD - 2 * N)))
    w_taps = w_taps.astype(jnp.float32)
    b_pad = jnp.pad(b_off.astype(jnp.float32),
                    (0, OFFPAD - 2 * N)).reshape(1, OFFPAD)
    wf = jnp.transpose(w_conv, (2, 3, 1, 0)).reshape(N, 1, C, outc)
    wf4 = jnp.broadcast_to(wf, (N, 4, C, outc)).reshape(N * 4 * C, outc)
    wf4 = wf4.astype(jnp.float32)
    j_i = jnp.arange(64)[:, None]
    m_i = jnp.arange(N * 4 * C)[None, :]
    n_of = m_i // (4 * C)
    k_of = (m_i % (4 * C)) // C
    emat = (j_i == (k_of * 16 + n_of)).astype(jnp.float32)

    cost = pl.CostEstimate(
        flops=2 * B * HW * C * (9 * OFFPAD + N * outc),
        transcendentals=0,
        bytes_accessed=4 * B * (Hp * Wp * C + HW * outc),
    )

    out = pl.pallas_call(
        functools.partial(_fused_deform_kernel, H=H, W=W, C=C, OUTC=outc,
                          UNROLL=UNROLL),
        out_shape=jax.ShapeDtypeStruct((B, HW, outc), jnp.float32),
        grid=(B,),
        in_specs=[
            pl.BlockSpec((1, Hp, Wp, C), lambda b: (b, 0, 0, 0)),
            pl.BlockSpec((9, C, OFFPAD), lambda b: (0, 0, 0)),
            pl.BlockSpec((1, OFFPAD), lambda b: (0, 0)),
            pl.BlockSpec((64, N * 4 * C), lambda b: (0, 0)),
            pl.BlockSpec((N * 4 * C, outc), lambda b: (0, 0)),
        ],
        out_specs=pl.BlockSpec((1, HW, outc), lambda b: (b, 0, 0)),
        scratch_shapes=[
            pltpu.VMEM((P, 1, 4 * C), jnp.float32),
            pltpu.VMEM((HW, OFFPAD), jnp.float32),
            pltpu.VMEM((16, HW), jnp.int32),
            pltpu.VMEM((HW, 64), jnp.float32),
            pltpu.VMEM((HW // UNROLL, UNROLL, N * 4 * C), jnp.float32),
            pltpu.SMEM((N * HW,), jnp.int32),
            pltpu.SemaphoreType.DMA,
        ],
        compiler_params=pltpu.CompilerParams(
            dimension_semantics=("parallel",),
            vmem_limit_bytes=100 * 1024 * 1024,
        ),
        cost_estimate=cost,
    )(xp, w_taps, b_pad, emat, wf4)

    out = out.reshape(B, H, W, outc)
    return jnp.transpose(out, (0, 3, 1, 2))


def kernel(x, w_off, b_off, w_conv):
    return _deform_conv2d(x, w_off, b_off, w_conv)


# speedup vs baseline: 2.9996x; 1.0467x over previous
import functools

import jax
import jax.numpy as jnp
from jax import lax
from jax.experimental import pallas as pl
from jax.experimental.pallas import tpu as pltpu


def _fused_deform_kernel(xp_ref, wt_ref, b_ref, e_ref, wf_ref, out_ref,
                         x4_ref, off_ref, idxT_ref, gw_ref, g_ref,
                         idx_smem, dma_sem, *, H, W, C, OUTC, UNROLL):
    Hp, Wp = H + 2, W + 2
    P = Hp * Wp
    HW = H * W
    N = 9
    f32 = jnp.float32

    x = xp_ref[0]
    acc = jnp.zeros((HW, off_ref.shape[1]), f32)
    for kr in range(3):
        for kc in range(3):
            patch = x[kr:kr + H, kc:kc + W, :].reshape(HW, C)
            acc = acc + jnp.dot(patch, wt_ref[kr * 3 + kc],
                                preferred_element_type=f32)
    off_ref[...] = acc + b_ref[...]

    xflat = x.reshape(P, C)
    x4_ref[:, 0, 0:C] = xflat
    x4_ref[0:P - 1, 0, C:2 * C] = xflat[1:P]
    x4_ref[0:P - Wp, 0, 2 * C:3 * C] = xflat[Wp:P]
    x4_ref[0:P - Wp - 1, 0, 3 * C:4 * C] = xflat[Wp + 1:P]

    off_r = off_ref[:, 0:N]
    off_c = off_ref[:, N:2 * N]
    t_idx = lax.broadcasted_iota(jnp.int32, (HW, 1), 0)
    h_i = t_idx // W
    w_i = t_idx - h_i * W
    n_i = lax.broadcasted_iota(jnp.int32, (1, N), 1)
    pn_r = (n_i // 3 - 1).astype(f32)
    pn_c = (n_i % 3 - 1).astype(f32)

    p_r = h_i.astype(f32) + 1.0 + pn_r + off_r
    p_c = w_i.astype(f32) + 1.0 + pn_c + off_c
    fr = jnp.floor(p_r)
    fc = jnp.floor(p_c)
    pr = jnp.clip(p_r, 0.0, Hp - 1.0)
    pc = jnp.clip(p_c, 0.0, Wp - 1.0)
    q0r = jnp.clip(fr, 0.0, Hp - 1.0)
    q1r = jnp.clip(fr + 1.0, 0.0, Hp - 1.0)
    q0c = jnp.clip(fc, 0.0, Wp - 1.0)
    q1c = jnp.clip(fc + 1.0, 0.0, Wp - 1.0)
    r0 = jnp.clip(fr, 0.0, Hp - 2.0)
    c0 = jnp.clip(fc, 0.0, Wp - 2.0)

    a0r = 1.0 + (q0r - pr)
    a1r = 1.0 - (q1r - pr)
    a0c = 1.0 + (q0c - pc)
    a1c = 1.0 - (q1c - pc)
    zero = jnp.zeros_like(a0r)
    w_top = (jnp.where(q0r == r0, a0r, zero)
             + jnp.where(q1r == r0, a1r, zero))
    w_bot = (jnp.where(q0r == r0 + 1.0, a0r, zero)
             + jnp.where(q1r == r0 + 1.0, a1r, zero))
    w_lft = (jnp.where(q0c == c0, a0c, zero)
             + jnp.where(q1c == c0, a1c, zero))
    w_rgt = (jnp.where(q0c == c0 + 1.0, a0c, zero)
             + jnp.where(q1c == c0 + 1.0, a1c, zero))

    gw_ref[...] = jnp.zeros(gw_ref.shape, f32)
    gw_ref[:, 0:N] = w_top * w_lft
    gw_ref[:, 16:16 + N] = w_top * w_rgt
    gw_ref[:, 32:32 + N] = w_bot * w_lft
    gw_ref[:, 48:48 + N] = w_bot * w_rgt

    idx = (r0 * Wp + c0).astype(jnp.int32)
    idxT_ref[0:N, :] = jnp.transpose(idx)
    for nn in range(N):
        pltpu.make_async_copy(idxT_ref.at[nn],
                              idx_smem.at[pl.ds(nn * HW, HW)], dma_sem).start()
    for nn in range(N):
        pltpu.make_async_copy(idxT_ref.at[nn],
                              idx_smem.at[pl.ds(nn * HW, HW)], dma_sem).wait()

    def gbody(j, _):
        tb = j * UNROLL
        for n in range(N):
            vals = [x4_ref[idx_smem[n * HW + tb + u], 0, :]
                    for u in range(UNROLL)]
            for u in range(UNROLL):
                g_ref[j, u, n * 4 * C:(n + 1) * 4 * C] = vals[u]
        return 0
    lax.fori_loop(0, HW // UNROLL, gbody, 0)

    gww = jnp.dot(gw_ref[...], e_ref[...], preferred_element_type=f32)
    gwide = g_ref[...].reshape(HW, N * 4 * C)
    out_ref[0] = jnp.dot(gww * gwide, wf_ref[...],
                         preferred_element_type=f32)


def _deform_conv2d(x, w_off, b_off, w_conv):
    B, C, H, W = x.shape
    Hp, Wp = H + 2, W + 2
    P = Hp * Wp
    HW = H * W
    N = 9
    outc = w_conv.shape[0]
    OFFPAD = 128
    UNROLL = 32

    x_nhwc = jnp.transpose(x, (0, 2, 3, 1)).astype(jnp.float32)
    xp = jnp.pad(x_nhwc, ((0, 0), (1, 1), (1, 1), (0, 0)))

    w_taps = jnp.transpose(w_off, (2, 3, 1, 0)).reshape(9, C, 2 * N)
    w_taps = jnp.pad(w_taps, ((0, 0), (0, 0), (0, OFFPAD - 2 * N)))
    w_taps = w_taps.astype(jnp.float32)
    b_pad = jnp.pad(b_off.astype(jnp.float32),
                    (0, OFFPAD - 2 * N)).reshape(1, OFFPAD)
    wf = jnp.transpose(w_conv, (2, 3, 1, 0)).reshape(N, 1, C, outc)
    wf4 = jnp.broadcast_to(wf, (N, 4, C, outc)).reshape(N * 4 * C, outc)
    wf4 = wf4.astype(jnp.float32)
    j_i = jnp.arange(64)[:, None]
    m_i = jnp.arange(N * 4 * C)[None, :]
    n_of = m_i // (4 * C)
    k_of = (m_i % (4 * C)) // C
    emat = (j_i == (k_of * 16 + n_of)).astype(jnp.float32)

    cost = pl.CostEstimate(
        flops=2 * B * HW * C * (9 * OFFPAD + N * outc),
        transcendentals=0,
        bytes_accessed=4 * B * (Hp * Wp * C + HW * outc),
    )

    out = pl.pallas_call(
        functools.partial(_fused_deform_kernel, H=H, W=W, C=C, OUTC=outc,
                          UNROLL=UNROLL),
        out_shape=jax.ShapeDtypeStruct((B, HW, outc), jnp.float32),
        grid=(B,),
        in_specs=[
            pl.BlockSpec((1, Hp, Wp, C), lambda b: (b, 0, 0, 0)),
            pl.BlockSpec((9, C, OFFPAD), lambda b: (0, 0, 0)),
            pl.BlockSpec((1, OFFPAD), lambda b: (0, 0)),
            pl.BlockSpec((64, N * 4 * C), lambda b: (0, 0)),
            pl.BlockSpec((N * 4 * C, outc), lambda b: (0, 0)),
        ],
        out_specs=pl.BlockSpec((1, HW, outc), lambda b: (b, 0, 0)),
        scratch_shapes=[
            pltpu.VMEM((P, 1, 4 * C), jnp.float32),
            pltpu.VMEM((HW, OFFPAD), jnp.float32),
            pltpu.VMEM((16, HW), jnp.int32),
            pltpu.VMEM((HW, 64), jnp.float32),
            pltpu.VMEM((HW // UNROLL, UNROLL, N * 4 * C), jnp.float32),
            pltpu.SMEM((N * HW,), jnp.int32),
            pltpu.SemaphoreType.DMA,
        ],
        compiler_params=pltpu.CompilerParams(
            dimension_semantics=("parallel",),
            vmem_limit_bytes=100 * 1024 * 1024,
        ),
        cost_estimate=cost,
    )(xp, w_taps, b_pad, emat, wf4)

    out = out.reshape(B, H, W, outc)
    return jnp.transpose(out, (0, 3, 1, 2))


def kernel(x, w_off, b_off, w_conv):
    return _deform_conv2d(x, w_off, b_off, w_conv)


# speedup vs baseline: 3.0119x; 1.0041x over previous
import functools

import jax
import jax.numpy as jnp
from jax import lax
from jax.experimental import pallas as pl
from jax.experimental.pallas import tpu as pltpu


def _fused_deform_kernel(xp_ref, wt_ref, b_ref, e_ref, wf_ref, out_ref,
                         x4_ref, off_ref, idxT_ref, gw_ref, g_ref,
                         idx_smem, dma_sem, *, H, W, C, OUTC, UNROLL):
    Hp, Wp = H + 2, W + 2
    P = Hp * Wp
    HW = H * W
    N = 9
    f32 = jnp.float32

    x = xp_ref[0]
    acc = jnp.zeros((HW, off_ref.shape[1]), f32)
    for kr in range(3):
        for kc in range(3):
            patch = x[kr:kr + H, kc:kc + W, :].reshape(HW, C)
            acc = acc + jnp.dot(patch, wt_ref[kr * 3 + kc],
                                preferred_element_type=f32)
    off_ref[...] = acc + b_ref[...]

    off_r = off_ref[:, 0:N]
    off_c = off_ref[:, N:2 * N]
    t_idx = lax.broadcasted_iota(jnp.int32, (HW, 1), 0)
    h_i = t_idx // W
    w_i = t_idx - h_i * W
    n_i = lax.broadcasted_iota(jnp.int32, (1, N), 1)
    pn_r = (n_i // 3 - 1).astype(f32)
    pn_c = (n_i % 3 - 1).astype(f32)

    p_r = h_i.astype(f32) + 1.0 + pn_r + off_r
    p_c = w_i.astype(f32) + 1.0 + pn_c + off_c
    fr = jnp.floor(p_r)
    fc = jnp.floor(p_c)
    pr = jnp.clip(p_r, 0.0, Hp - 1.0)
    pc = jnp.clip(p_c, 0.0, Wp - 1.0)
    q0r = jnp.clip(fr, 0.0, Hp - 1.0)
    q1r = jnp.clip(fr + 1.0, 0.0, Hp - 1.0)
    q0c = jnp.clip(fc, 0.0, Wp - 1.0)
    q1c = jnp.clip(fc + 1.0, 0.0, Wp - 1.0)
    r0 = jnp.clip(fr, 0.0, Hp - 2.0)
    c0 = jnp.clip(fc, 0.0, Wp - 2.0)

    a0r = 1.0 + (q0r - pr)
    a1r = 1.0 - (q1r - pr)
    a0c = 1.0 + (q0c - pc)
    a1c = 1.0 - (q1c - pc)
    zero = jnp.zeros_like(a0r)
    w_top = (jnp.where(q0r == r0, a0r, zero)
             + jnp.where(q1r == r0, a1r, zero))
    w_bot = (jnp.where(q0r == r0 + 1.0, a0r, zero)
             + jnp.where(q1r == r0 + 1.0, a1r, zero))
    w_lft = (jnp.where(q0c == c0, a0c, zero)
             + jnp.where(q1c == c0, a1c, zero))
    w_rgt = (jnp.where(q0c == c0 + 1.0, a0c, zero)
             + jnp.where(q1c == c0 + 1.0, a1c, zero))

    gw_ref[...] = jnp.zeros(gw_ref.shape, f32)
    gw_ref[:, 0:N] = w_top * w_lft
    gw_ref[:, 16:16 + N] = w_top * w_rgt
    gw_ref[:, 32:32 + N] = w_bot * w_lft
    gw_ref[:, 48:48 + N] = w_bot * w_rgt

    idx = (r0 * Wp + c0).astype(jnp.int32)
    idxT_ref[0:N, :] = jnp.transpose(idx)
    for nn in range(N):
        pltpu.make_async_copy(idxT_ref.at[nn],
                              idx_smem.at[pl.ds(nn * HW, HW)], dma_sem).start()

    xflat = x.reshape(P, C)
    x4_ref[:, 0, 0:C] = xflat
    x4_ref[0:P - 1, 0, C:2 * C] = xflat[1:P]
    x4_ref[0:P - Wp, 0, 2 * C:3 * C] = xflat[Wp:P]
    x4_ref[0:P - Wp - 1, 0, 3 * C:4 * C] = xflat[Wp + 1:P]

    for nn in range(N):
        pltpu.make_async_copy(idxT_ref.at[nn],
                              idx_smem.at[pl.ds(nn * HW, HW)], dma_sem).wait()

    def gbody(j, _):
        tb = j * UNROLL
        for n in range(N):
            vals = [x4_ref[idx_smem[n * HW + tb + u], 0, :]
                    for u in range(UNROLL)]
            for u in range(UNROLL):
                g_ref[j, u, n * 4 * C:(n + 1) * 4 * C] = vals[u]
        return 0
    lax.fori_loop(0, HW // UNROLL, gbody, 0)

    gww = jnp.dot(gw_ref[...], e_ref[...], preferred_element_type=f32)
    gwide = g_ref[...].reshape(HW, N * 4 * C)
    out_ref[0] = jnp.dot(gww * gwide, wf_ref[...],
                         preferred_element_type=f32)


def _deform_conv2d(x, w_off, b_off, w_conv):
    B, C, H, W = x.shape
    Hp, Wp = H + 2, W + 2
    P = Hp * Wp
    HW = H * W
    N = 9
    outc = w_conv.shape[0]
    OFFPAD = 128
    UNROLL = 32

    x_nhwc = jnp.transpose(x, (0, 2, 3, 1)).astype(jnp.float32)
    xp = jnp.pad(x_nhwc, ((0, 0), (1, 1), (1, 1), (0, 0)))

    w_taps = jnp.transpose(w_off, (2, 3, 1, 0)).reshape(9, C, 2 * N)
    w_taps = jnp.pad(w_taps, ((0, 0), (0, 0), (0, OFFPAD - 2 * N)))
    w_taps = w_taps.astype(jnp.float32)
    b_pad = jnp.pad(b_off.astype(jnp.float32),
                    (0, OFFPAD - 2 * N)).reshape(1, OFFPAD)
    wf = jnp.transpose(w_conv, (2, 3, 1, 0)).reshape(N, 1, C, outc)
    wf4 = jnp.broadcast_to(wf, (N, 4, C, outc)).reshape(N * 4 * C, outc)
    wf4 = wf4.astype(jnp.float32)
    j_i = jnp.arange(64)[:, None]
    m_i = jnp.arange(N * 4 * C)[None, :]
    n_of = m_i // (4 * C)
    k_of = (m_i % (4 * C)) // C
    emat = (j_i == (k_of * 16 + n_of)).astype(jnp.float32)

    cost = pl.CostEstimate(
        flops=2 * B * HW * C * (9 * OFFPAD + N * outc),
        transcendentals=0,
        bytes_accessed=4 * B * (Hp * Wp * C + HW * outc),
    )

    out = pl.pallas_call(
        functools.partial(_fused_deform_kernel, H=H, W=W, C=C, OUTC=outc,
                          UNROLL=UNROLL),
        out_shape=jax.ShapeDtypeStruct((B, HW, outc), jnp.float32),
        grid=(B,),
        in_specs=[
            pl.BlockSpec((1, Hp, Wp, C), lambda b: (b, 0, 0, 0)),
            pl.BlockSpec((9, C, OFFPAD), lambda b: (0, 0, 0)),
            pl.BlockSpec((1, OFFPAD), lambda b: (0, 0)),
            pl.BlockSpec((64, N * 4 * C), lambda b: (0, 0)),
            pl.BlockSpec((N * 4 * C, outc), lambda b: (0, 0)),
        ],
        out_specs=pl.BlockSpec((1, HW, outc), lambda b: (b, 0, 0)),
        scratch_shapes=[
            pltpu.VMEM((P, 1, 4 * C), jnp.float32),
            pltpu.VMEM((HW, OFFPAD), jnp.float32),
            pltpu.VMEM((16, HW), jnp.int32),
            pltpu.VMEM((HW, 64), jnp.float32),
            pltpu.VMEM((HW // UNROLL, UNROLL, N * 4 * C), jnp.float32),
            pltpu.SMEM((N * HW,), jnp.int32),
            pltpu.SemaphoreType.DMA,
        ],
        compiler_params=pltpu.CompilerParams(
            dimension_semantics=("parallel",),
            vmem_limit_bytes=100 * 1024 * 1024,
        ),
        cost_estimate=cost,
    )(xp, w_taps, b_pad, emat, wf4)

    out = out.reshape(B, H, W, outc)
    return jnp.transpose(out, (0, 3, 1, 2))


def kernel(x, w_off, b_off, w_conv):
    return _deform_conv2d(x, w_off, b_off, w_conv)


# speedup vs baseline: 3.0211x; 1.0031x over previous
import functools

import jax
import jax.numpy as jnp
from jax import lax
from jax.experimental import pallas as pl
from jax.experimental.pallas import tpu as pltpu


def _fused_deform_kernel(xp_ref, wt_ref, b_ref, e_ref, wf_ref, out_ref,
                         x4_ref, off_ref, idxT_ref, gw_ref, g_ref,
                         idx_smem, dma_sem, *, H, W, C, OUTC, UNROLL):
    Hp, Wp = H + 2, W + 2
    P = Hp * Wp
    HW = H * W
    N = 9
    f32 = jnp.float32

    x = xp_ref[0]
    acc = jnp.zeros((HW, off_ref.shape[1]), f32)
    for kr in range(3):
        for kc in range(3):
            patch = x[kr:kr + H, kc:kc + W, :].reshape(HW, C)
            acc = acc + jnp.dot(patch, wt_ref[kr * 3 + kc],
                                preferred_element_type=f32)
    off_ref[...] = acc + b_ref[...]

    off_r = off_ref[:, 0:N]
    off_c = off_ref[:, N:2 * N]
    t_idx = lax.broadcasted_iota(jnp.int32, (HW, 1), 0)
    h_i = t_idx // W
    w_i = t_idx - h_i * W
    n_i = lax.broadcasted_iota(jnp.int32, (1, N), 1)
    pn_r = (n_i // 3 - 1).astype(f32)
    pn_c = (n_i % 3 - 1).astype(f32)

    p_r = h_i.astype(f32) + 1.0 + pn_r + off_r
    p_c = w_i.astype(f32) + 1.0 + pn_c + off_c
    fr = jnp.floor(p_r)
    fc = jnp.floor(p_c)
    pr = jnp.clip(p_r, 0.0, Hp - 1.0)
    pc = jnp.clip(p_c, 0.0, Wp - 1.0)
    q0r = jnp.clip(fr, 0.0, Hp - 1.0)
    q1r = jnp.clip(fr + 1.0, 0.0, Hp - 1.0)
    q0c = jnp.clip(fc, 0.0, Wp - 1.0)
    q1c = jnp.clip(fc + 1.0, 0.0, Wp - 1.0)
    r0 = jnp.clip(fr, 0.0, Hp - 2.0)
    c0 = jnp.clip(fc, 0.0, Wp - 2.0)

    a0r = 1.0 + (q0r - pr)
    a1r = 1.0 - (q1r - pr)
    a0c = 1.0 + (q0c - pc)
    a1c = 1.0 - (q1c - pc)
    zero = jnp.zeros_like(a0r)
    w_top = (jnp.where(q0r == r0, a0r, zero)
             + jnp.where(q1r == r0, a1r, zero))
    w_bot = (jnp.where(q0r == r0 + 1.0, a0r, zero)
             + jnp.where(q1r == r0 + 1.0, a1r, zero))
    w_lft = (jnp.where(q0c == c0, a0c, zero)
             + jnp.where(q1c == c0, a1c, zero))
    w_rgt = (jnp.where(q0c == c0 + 1.0, a0c, zero)
             + jnp.where(q1c == c0 + 1.0, a1c, zero))

    gw_ref[...] = jnp.zeros(gw_ref.shape, f32)
    gw_ref[:, 0:N] = w_top * w_lft
    gw_ref[:, 16:16 + N] = w_top * w_rgt
    gw_ref[:, 32:32 + N] = w_bot * w_lft
    gw_ref[:, 48:48 + N] = w_bot * w_rgt

    idx = (r0 * Wp + c0).astype(jnp.int32)
    idxT_ref[0:N, :] = jnp.transpose(idx)
    for nn in range(N):
        pltpu.make_async_copy(idxT_ref.at[nn],
                              idx_smem.at[pl.ds(nn * HW, HW)], dma_sem).start()

    xflat = x.reshape(P, C)
    x4_ref[:, 0, 0:C] = xflat
    x4_ref[0:P - 1, 0, C:2 * C] = xflat[1:P]
    x4_ref[0:P - Wp, 0, 2 * C:3 * C] = xflat[Wp:P]
    x4_ref[0:P - Wp - 1, 0, 3 * C:4 * C] = xflat[Wp + 1:P]

    for nn in range(N):
        pltpu.make_async_copy(idxT_ref.at[nn],
                              idx_smem.at[pl.ds(nn * HW, HW)], dma_sem).wait()

    def gbody(j, _):
        tb = j * UNROLL
        for n in range(N):
            vals = [x4_ref[idx_smem[n * HW + tb + u], 0, :]
                    for u in range(UNROLL)]
            for u in range(UNROLL):
                g_ref[j, u, n * 4 * C:(n + 1) * 4 * C] = vals[u]
        return 0
    lax.fori_loop(0, HW // UNROLL, gbody, 0)

    gww = jnp.dot(gw_ref[...], e_ref[...], preferred_element_type=f32)
    gwide = g_ref[...].reshape(HW, N * 4 * C)
    out_ref[0] = jnp.dot(gww * gwide, wf_ref[...],
                         preferred_element_type=f32)


def _deform_conv2d(x, w_off, b_off, w_conv):
    B, C, H, W = x.shape
    Hp, Wp = H + 2, W + 2
    P = Hp * Wp
    HW = H * W
    N = 9
    outc = w_conv.shape[0]
    OFFPAD = 128
    UNROLL = 64

    x_nhwc = jnp.transpose(x, (0, 2, 3, 1)).astype(jnp.float32)
    xp = jnp.pad(x_nhwc, ((0, 0), (1, 1), (1, 1), (0, 0)))

    w_taps = jnp.transpose(w_off, (2, 3, 1, 0)).reshape(9, C, 2 * N)
    w_taps = jnp.pad(w_taps, ((0, 0), (0, 0), (0, OFFPAD - 2 * N)))
    w_taps = w_taps.astype(jnp.float32)
    b_pad = jnp.pad(b_off.astype(jnp.float32),
                    (0, OFFPAD - 2 * N)).reshape(1, OFFPAD)
    wf = jnp.transpose(w_conv, (2, 3, 1, 0)).reshape(N, 1, C, outc)
    wf4 = jnp.broadcast_to(wf, (N, 4, C, outc)).reshape(N * 4 * C, outc)
    wf4 = wf4.astype(jnp.float32)
    j_i = jnp.arange(64)[:, None]
    m_i = jnp.arange(N * 4 * C)[None, :]
    n_of = m_i // (4 * C)
    k_of = (m_i % (4 * C)) // C
    emat = (j_i == (k_of * 16 + n_of)).astype(jnp.float32)

    cost = pl.CostEstimate(
        flops=2 * B * HW * C * (9 * OFFPAD + N * outc),
        transcendentals=0,
        bytes_accessed=4 * B * (Hp * Wp * C + HW * outc),
    )

    out = pl.pallas_call(
        functools.partial(_fused_deform_kernel, H=H, W=W, C=C, OUTC=outc,
                          UNROLL=UNROLL),
        out_shape=jax.ShapeDtypeStruct((B, HW, outc), jnp.float32),
        grid=(B,),
        in_specs=[
            pl.BlockSpec((1, Hp, Wp, C), lambda b: (b, 0, 0, 0)),
            pl.BlockSpec((9, C, OFFPAD), lambda b: (0, 0, 0)),
            pl.BlockSpec((1, OFFPAD), lambda b: (0, 0)),
            pl.BlockSpec((64, N * 4 * C), lambda b: (0, 0)),
            pl.BlockSpec((N * 4 * C, outc), lambda b: (0, 0)),
        ],
        out_specs=pl.BlockSpec((1, HW, outc), lambda b: (b, 0, 0)),
        scratch_shapes=[
            pltpu.VMEM((P, 1, 4 * C), jnp.float32),
            pltpu.VMEM((HW, OFFPAD), jnp.float32),
            pltpu.VMEM((16, HW), jnp.int32),
            pltpu.VMEM((HW, 64), jnp.float32),
            pltpu.VMEM((HW // UNROLL, UNROLL, N * 4 * C), jnp.float32),
            pltpu.SMEM((N * HW,), jnp.int32),
            pltpu.SemaphoreType.DMA,
        ],
        compiler_params=pltpu.CompilerParams(
            dimension_semantics=("parallel",),
            vmem_limit_bytes=100 * 1024 * 1024,
        ),
        cost_estimate=cost,
    )(xp, w_taps, b_pad, emat, wf4)

    out = out.reshape(B, H, W, outc)
    return jnp.transpose(out, (0, 3, 1, 2))


def kernel(x, w_off, b_off, w_conv):
    return _deform_conv2d(x, w_off, b_off, w_conv)


# speedup vs baseline: 3.0225x; 1.0005x over previous
import functools

import jax
import jax.numpy as jnp
from jax import lax
from jax.experimental import pallas as pl
from jax.experimental.pallas import tpu as pltpu


def _fused_deform_kernel(xp_ref, wt_ref, b_ref, e_ref, wf_ref, out_ref,
                         x4_ref, off_ref, idxT_ref, gw_ref, g_ref,
                         idx_smem, dma_sem, *, H, W, C, OUTC, UNROLL):
    Hp, Wp = H + 2, W + 2
    P = Hp * Wp
    HW = H * W
    N = 9
    f32 = jnp.float32

    x = xp_ref[0]
    acc = jnp.zeros((HW, off_ref.shape[1]), f32)
    for kr in range(3):
        for kc in range(3):
            patch = x[kr:kr + H, kc:kc + W, :].reshape(HW, C)
            acc = acc + jnp.dot(patch, wt_ref[kr * 3 + kc],
                                preferred_element_type=f32)
    off_ref[...] = acc + b_ref[...]

    off_r = off_ref[:, 0:N]
    off_c = off_ref[:, N:2 * N]
    t_idx = lax.broadcasted_iota(jnp.int32, (HW, 1), 0)
    h_i = t_idx // W
    w_i = t_idx - h_i * W
    n_i = lax.broadcasted_iota(jnp.int32, (1, N), 1)
    pn_r = (n_i // 3 - 1).astype(f32)
    pn_c = (n_i % 3 - 1).astype(f32)

    p_r = h_i.astype(f32) + 1.0 + pn_r + off_r
    p_c = w_i.astype(f32) + 1.0 + pn_c + off_c
    fr = jnp.floor(p_r)
    fc = jnp.floor(p_c)
    pr = jnp.clip(p_r, 0.0, Hp - 1.0)
    pc = jnp.clip(p_c, 0.0, Wp - 1.0)
    q0r = jnp.clip(fr, 0.0, Hp - 1.0)
    q1r = jnp.clip(fr + 1.0, 0.0, Hp - 1.0)
    q0c = jnp.clip(fc, 0.0, Wp - 1.0)
    q1c = jnp.clip(fc + 1.0, 0.0, Wp - 1.0)
    r0 = jnp.clip(fr, 0.0, Hp - 2.0)
    c0 = jnp.clip(fc, 0.0, Wp - 2.0)

    a0r = 1.0 + (q0r - pr)
    a1r = 1.0 - (q1r - pr)
    a0c = 1.0 + (q0c - pc)
    a1c = 1.0 - (q1c - pc)
    zero = jnp.zeros_like(a0r)
    w_top = (jnp.where(q0r == r0, a0r, zero)
             + jnp.where(q1r == r0, a1r, zero))
    w_bot = (jnp.where(q0r == r0 + 1.0, a0r, zero)
             + jnp.where(q1r == r0 + 1.0, a1r, zero))
    w_lft = (jnp.where(q0c == c0, a0c, zero)
             + jnp.where(q1c == c0, a1c, zero))
    w_rgt = (jnp.where(q0c == c0 + 1.0, a0c, zero)
             + jnp.where(q1c == c0 + 1.0, a1c, zero))

    gw_ref[...] = jnp.zeros(gw_ref.shape, f32)
    gw_ref[:, 0:N] = w_top * w_lft
    gw_ref[:, 16:16 + N] = w_top * w_rgt
    gw_ref[:, 32:32 + N] = w_bot * w_lft
    gw_ref[:, 48:48 + N] = w_bot * w_rgt

    idx = (r0 * Wp + c0).astype(jnp.int32)
    idxT_ref[0:N, :] = jnp.transpose(idx)
    for nn in range(N):
        pltpu.make_async_copy(idxT_ref.at[nn],
                              idx_smem.at[pl.ds(nn * HW, HW)], dma_sem).start()

    xflat = x.reshape(P, C)
    x4_ref[:, 0, 0:C] = xflat
    x4_ref[0:P - 1, 0, C:2 * C] = xflat[1:P]
    x4_ref[0:P - Wp, 0, 2 * C:3 * C] = xflat[Wp:P]
    x4_ref[0:P - Wp - 1, 0, 3 * C:4 * C] = xflat[Wp + 1:P]

    for nn in range(N):
        pltpu.make_async_copy(idxT_ref.at[nn],
                              idx_smem.at[pl.ds(nn * HW, HW)], dma_sem).wait()

    def gbody(j, _):
        tb = j * UNROLL
        for n in range(N):
            vals = [x4_ref[idx_smem[n * HW + tb + u], 0, :]
                    for u in range(UNROLL)]
            for u in range(UNROLL):
                g_ref[j, u, n * 4 * C:(n + 1) * 4 * C] = vals[u]
        return 0
    lax.fori_loop(0, HW // UNROLL, gbody, 0)

    gww = jnp.dot(gw_ref[...].astype(jnp.bfloat16), e_ref[...],
                  preferred_element_type=f32)
    gwide = g_ref[...].reshape(HW, N * 4 * C)
    out_ref[0] = jnp.dot((gww * gwide).astype(jnp.bfloat16), wf_ref[...],
                         preferred_element_type=f32)


def _deform_conv2d(x, w_off, b_off, w_conv):
    B, C, H, W = x.shape
    Hp, Wp = H + 2, W + 2
    P = Hp * Wp
    HW = H * W
    N = 9
    outc = w_conv.shape[0]
    OFFPAD = 128
    UNROLL = 64

    x_nhwc = jnp.transpose(x, (0, 2, 3, 1)).astype(jnp.float32)
    xp = jnp.pad(x_nhwc, ((0, 0), (1, 1), (1, 1), (0, 0)))

    w_taps = jnp.transpose(w_off, (2, 3, 1, 0)).reshape(9, C, 2 * N)
    w_taps = jnp.pad(w_taps, ((0, 0), (0, 0), (0, OFFPAD - 2 * N)))
    w_taps = w_taps.astype(jnp.float32)
    b_pad = jnp.pad(b_off.astype(jnp.float32),
                    (0, OFFPAD - 2 * N)).reshape(1, OFFPAD)
    wf = jnp.transpose(w_conv, (2, 3, 1, 0)).reshape(N, 1, C, outc)
    wf4 = jnp.broadcast_to(wf, (N, 4, C, outc)).reshape(N * 4 * C, outc)
    wf4 = wf4.astype(jnp.bfloat16)
    j_i = jnp.arange(64)[:, None]
    m_i = jnp.arange(N * 4 * C)[None, :]
    n_of = m_i // (4 * C)
    k_of = (m_i % (4 * C)) // C
    emat = (j_i == (k_of * 16 + n_of)).astype(jnp.bfloat16)

    cost = pl.CostEstimate(
        flops=2 * B * HW * C * (9 * OFFPAD + N * outc),
        transcendentals=0,
        bytes_accessed=4 * B * (Hp * Wp * C + HW * outc),
    )

    out = pl.pallas_call(
        functools.partial(_fused_deform_kernel, H=H, W=W, C=C, OUTC=outc,
                          UNROLL=UNROLL),
        out_shape=jax.ShapeDtypeStruct((B, HW, outc), jnp.float32),
        grid=(B,),
        in_specs=[
            pl.BlockSpec((1, Hp, Wp, C), lambda b: (b, 0, 0, 0)),
            pl.BlockSpec((9, C, OFFPAD), lambda b: (0, 0, 0)),
            pl.BlockSpec((1, OFFPAD), lambda b: (0, 0)),
            pl.BlockSpec((64, N * 4 * C), lambda b: (0, 0)),
            pl.BlockSpec((N * 4 * C, outc), lambda b: (0, 0)),
        ],
        out_specs=pl.BlockSpec((1, HW, outc), lambda b: (b, 0, 0)),
        scratch_shapes=[
            pltpu.VMEM((P, 1, 4 * C), jnp.float32),
            pltpu.VMEM((HW, OFFPAD), jnp.float32),
            pltpu.VMEM((16, HW), jnp.int32),
            pltpu.VMEM((HW, 64), jnp.float32),
            pltpu.VMEM((HW // UNROLL, UNROLL, N * 4 * C), jnp.float32),
            pltpu.SMEM((N * HW,), jnp.int32),
            pltpu.SemaphoreType.DMA,
        ],
        compiler_params=pltpu.CompilerParams(
            dimension_semantics=("parallel",),
            vmem_limit_bytes=100 * 1024 * 1024,
        ),
        cost_estimate=cost,
    )(xp, w_taps, b_pad, emat, wf4)

    out = out.reshape(B, H, W, outc)
    return jnp.transpose(out, (0, 3, 1, 2))


def kernel(x, w_off, b_off, w_conv):
    return _deform_conv2d(x, w_off, b_off, w_conv)
